# Initial kernel scaffold; baseline (speedup 1.0000x reference)
import jax
import jax.numpy as jnp
from jax.experimental import pallas as pl


def kernel(inputs, weight, biases, s0_rows, s0_cols, s0_vals, s1_rows, s1_cols, s1_vals):
    raise NotImplementedError("write your pallas kernel here")


# speedup vs baseline: 1.0370x; 1.0370x over previous
import jax
import jax.numpy as jnp
from jax.experimental import pallas as pl

N = 50000
E = 800000
B = 4
ISZ = 66
OUT = 64
NM = 5
FP = 288
TN = 512


def _mm_kernel(x_ref, w_ref, b_ref, o_ref):
    o_ref[...] = (
        jnp.dot(x_ref[...], w_ref[...], preferred_element_type=jnp.float32)
        + b_ref[...]
    )


def _matmul(x, w, bias_row):
    return pl.pallas_call(
        _mm_kernel,
        grid=(pl.cdiv(N, TN),),
        in_specs=[
            pl.BlockSpec((TN, NM * FP), lambda i: (i, 0)),
            pl.BlockSpec((NM * FP, B * OUT), lambda i: (0, 0)),
            pl.BlockSpec((1, B * OUT), lambda i: (0, 0)),
        ],
        out_specs=pl.BlockSpec((TN, B * OUT), lambda i: (i, 0)),
        out_shape=jax.ShapeDtypeStruct((N, B * OUT), jnp.float32),
    )(x, w, bias_row)


def _spmm(rows, cols, vals, mat):
    gathered = jnp.take(mat, cols, axis=0)
    return jax.ops.segment_sum(vals[:, None] * gathered, rows, num_segments=N)


def kernel(inputs, weight, biases, s0_rows, s0_cols, s0_vals, s1_rows, s1_cols, s1_vals):
    w = weight.reshape(ISZ, NM, OUT)
    w0, w1, w2, w3, w4 = (w[:, m] for m in range(NM))
    wm = jnp.stack([w0 - w2, w1 - w4, 2.0 * w2, w3, 2.0 * w4], axis=0)
    wm = jnp.pad(wm, ((0, 0), (0, FP // B - ISZ), (0, 0)))
    eye = jnp.eye(B, dtype=jnp.float32)
    wbig = wm[:, :, None, None, :] * eye[None, None, :, :, None]
    wbig = wbig.reshape(NM * FP, B * OUT)

    x = inputs.reshape(B, N, ISZ)
    x0 = jnp.transpose(x, (1, 2, 0)).reshape(N, ISZ * B)

    y1 = _spmm(s0_rows, s0_cols, s0_vals, x0)
    y2 = _spmm(s0_rows, s0_cols, s0_vals, y1)
    y3 = _spmm(s1_rows, s1_cols, s1_vals, y1)
    y4 = _spmm(s1_rows, s1_cols, s1_vals, y3)

    pad = ((0, 0), (0, FP - ISZ * B))
    xcat = jnp.concatenate([jnp.pad(m, pad) for m in (x0, y1, y2, y3, y4)], axis=1)

    bias_row = jnp.tile(biases, B)[None, :]
    out2 = _matmul(xcat, wbig, bias_row)
    return out2.reshape(N, B, OUT).transpose(1, 0, 2).reshape(B, N * OUT)
